# baseline (speedup 1.0000x reference)
import functools

import jax
import jax.numpy as jnp
from jax import lax
from jax.experimental import pallas as pl
from jax.experimental.pallas import tpu as pltpu
from jax.experimental.pallas import tpu_sc as plsc

_pc = pl.pallas_call

N_INST = 50000
N_FINAL = 1000
E_PREV = 800000
E_TOF = 50000
D_IN = 128
H = 64
HH = 32
QQ = 16
DE = 16
EH = 8
NL = 11

NP = 50176
FP = 1024
EPP = 819200
EPR = EPP // 128
ETP = 65536
ETR = ETP // 128
NPK = NP // 8

NC, NS = 2, 16
ROWS_PER_TILE = NP // NS
CHUNK = 128
GRP = 4
PREV_GROUPS = EPR // (NS * GRP)
TOF_GROUPS = ETR // (NS * GRP)


def _elu(x):
    return jnp.where(x > 0.0, x, jnp.exp(jnp.minimum(x, 0.0)) - 1.0)


def _enc_inst_body(x_ref, w_ref, b_ref, o_ref):
    y = _elu(jnp.dot(x_ref[...], w_ref[...],
                     preferred_element_type=jnp.float32) + b_ref[...])
    for q in range(4):
        o_ref[q] = y[:, q * QQ:(q + 1) * QQ]


def _enc_inst(xp, w, b):
    return _pc(
        _enc_inst_body,
        grid=(NP // 1024,),
        in_specs=[
            pl.BlockSpec((1024, D_IN), lambda i: (i, 0)),
            pl.BlockSpec((D_IN, H), lambda i: (0, 0)),
            pl.BlockSpec((1, H), lambda i: (0, 0)),
        ],
        out_specs=pl.BlockSpec((4, 1024, QQ), lambda i: (0, i, 0)),
        out_shape=jax.ShapeDtypeStruct((4, NP, QQ), jnp.float32),
    )(xp, w, b)


def _enc_edge_body(e_ref, wp_ref, bp_ref, ws_ref, bs_ref, o_ref):
    e = e_ref[...]
    one = jnp.ones((e.shape[0], 1), jnp.float32)
    pad = jnp.zeros((e.shape[0], 7), jnp.float32)
    yp = _elu(jnp.dot(e, wp_ref[...], preferred_element_type=jnp.float32)
              + bp_ref[...])
    ys = _elu(jnp.dot(e, ws_ref[...], preferred_element_type=jnp.float32)
              + bs_ref[...])
    o_ref[0] = jnp.concatenate([yp, one, pad], axis=1)
    o_ref[1] = jnp.concatenate([ys, one, pad], axis=1)


def _enc_edge(ep, wp, bp, ws, bs):
    return _pc(
        _enc_edge_body,
        grid=(EPP // 2048,),
        in_specs=[
            pl.BlockSpec((2048, DE), lambda i: (i, 0)),
            pl.BlockSpec((DE, EH), lambda i: (0, 0)),
            pl.BlockSpec((1, EH), lambda i: (0, 0)),
            pl.BlockSpec((DE, EH), lambda i: (0, 0)),
            pl.BlockSpec((1, EH), lambda i: (0, 0)),
        ],
        out_specs=pl.BlockSpec((2, 2048, DE), lambda i: (0, i, 0)),
        out_shape=jax.ShapeDtypeStruct((2, EPP, DE), jnp.float32),
    )(ep, wp, bp, ws, bs)


def _enc_tof_body(e_ref, w_ref, b_ref, o_ref):
    e = e_ref[...]
    y = _elu(jnp.dot(e, w_ref[...], preferred_element_type=jnp.float32)
             + b_ref[...])
    one = jnp.ones((e.shape[0], 1), jnp.float32)
    pad = jnp.zeros((e.shape[0], 7), jnp.float32)
    o_ref[...] = jnp.concatenate([y, one, pad], axis=1)


def _enc_tof(ep, w, b):
    return _pc(
        _enc_tof_body,
        grid=(ETP // 2048,),
        in_specs=[
            pl.BlockSpec((2048, DE), lambda i: (i, 0)),
            pl.BlockSpec((DE, EH), lambda i: (0, 0)),
            pl.BlockSpec((1, EH), lambda i: (0, 0)),
        ],
        out_specs=pl.BlockSpec((2048, DE), lambda i: (i, 0)),
        out_shape=jax.ShapeDtypeStruct((ETP, DE), jnp.float32),
    )(ep, w, b)


def _meta_body(es_ref, msel_ref, memp_ref, mems_ref, o_ref):
    dot = lambda a, b: jnp.dot(a, b, preferred_element_type=jnp.float32)
    es0, es1 = es_ref[0], es_ref[1]
    cnt0 = dot(es0, msel_ref[...])
    cnt1 = dot(es1, msel_ref[...])
    inv0 = 1.0 / jnp.maximum(cnt0, 1.0)
    inv1 = 1.0 / jnp.maximum(cnt1, 1.0)
    o_ref[0] = inv0
    o_ref[1] = inv1
    o_ref[2] = (cnt0 > 0.0).astype(jnp.float32)
    o_ref[3] = (cnt1 > 0.0).astype(jnp.float32)
    o_ref[4] = dot(es0 * inv0, memp_ref[...]) + dot(es1 * inv1, mems_ref[...])


def _meta_inst(esk, msel, memp, mems):
    return _pc(
        _meta_body,
        grid=(NPK // 1568,),
        in_specs=[
            pl.BlockSpec((2, 1568, 128), lambda i: (0, i, 0)),
            pl.BlockSpec((128, 128), lambda i: (0, 0)),
            pl.BlockSpec((128, 128), lambda i: (0, 0)),
            pl.BlockSpec((128, 128), lambda i: (0, 0)),
        ],
        out_specs=pl.BlockSpec((5, 1568, 128), lambda i: (0, i, 0)),
        out_shape=jax.ShapeDtypeStruct((5, NPK, 128), jnp.float32),
    )(esk, msel, memp, mems)


def _meta_tof_body(st_ref, o_ref):
    cnt = st_ref[:, EH:EH + 1]
    inv = 1.0 / jnp.maximum(cnt, 1.0)
    fl = (cnt > 0.0).astype(jnp.float32)
    z8 = jnp.zeros((cnt.shape[0], 8), jnp.float32)
    z14 = jnp.zeros((cnt.shape[0], 14), jnp.float32)
    o_ref[...] = jnp.concatenate([st_ref[:, :EH] * inv, z8, inv, fl, z14],
                                 axis=1)


def _meta_tof(st):
    return _pc(
        _meta_tof_body,
        grid=(1,),
        in_specs=[pl.BlockSpec((FP, DE), lambda i: (0, 0))],
        out_specs=pl.BlockSpec((FP, HH), lambda i: (0, 0)),
        out_shape=jax.ShapeDtypeStruct((FP, HH), jnp.float32),
    )(st)


def _tc_layer_body(x_ref, p_ref, s_ref, m_ref, w0_ref, w1_ref, wb_ref,
                   bt_ref, o_ref):
    dot = lambda a, b: jnp.dot(a, b, preferred_element_type=jnp.float32)
    invp, invs = m_ref[0], m_ref[1]
    flp, fls = m_ref[2], m_ref[3]
    em2 = m_ref[4]
    pm = [p_ref[qp] * invp for qp in range(4)]
    sm = [s_ref[qp] * invs for qp in range(4)]
    for q in range(4):
        u = dot(em2, wb_ref[q])
        for qp in range(4):
            u = u + dot(pm[qp], w0_ref[qp, q]) + dot(sm[qp], w1_ref[qp, q])
        u = u + flp * bt_ref[0, q][None, :] + fls * bt_ref[1, q][None, :]
        o_ref[q] = _elu(x_ref[q] + 0.5 * u)


def _tc_layer(xk, pk, sk, meta5, w0big, w1big, wbbig, btile):
    return _pc(
        _tc_layer_body,
        grid=(NPK // 1568,),
        in_specs=[
            pl.BlockSpec((4, 1568, 128), lambda i: (0, i, 0)),
            pl.BlockSpec((4, 1568, 128), lambda i: (0, i, 0)),
            pl.BlockSpec((4, 1568, 128), lambda i: (0, i, 0)),
            pl.BlockSpec((5, 1568, 128), lambda i: (0, i, 0)),
            pl.BlockSpec((4, 4, 128, 128), lambda i: (0, 0, 0, 0)),
            pl.BlockSpec((4, 4, 128, 128), lambda i: (0, 0, 0, 0)),
            pl.BlockSpec((4, 128, 128), lambda i: (0, 0, 0)),
            pl.BlockSpec((2, 4, 128), lambda i: (0, 0, 0)),
        ],
        out_specs=pl.BlockSpec((4, 1568, 128), lambda i: (0, i, 0)),
        out_shape=jax.ShapeDtypeStruct((4, NPK, 128), jnp.float32),
    )(xk, pk, sk, meta5, w0big, w1big, wbbig, btile)


def _tc_final_body(ff_ref, wf_ref, bf_ref, mt_ref, gw_ref, gb_ref,
                   r0_ref, rb0_ref, r1_ref, rb1_ref, r2_ref, rb2_ref,
                   *t_refs):
    t_refs, o_ref = t_refs[:-1], t_refs[-1]
    fin = _elu(jnp.dot(ff_ref[...], wf_ref[...],
                       preferred_element_type=jnp.float32) + bf_ref[...])
    m = mt_ref[...]
    emt = m[:, :EH]
    inv_t, fl_t = m[:, 16:17], m[:, 17:18]
    for l in range(NL):
        w = gw_ref[l]
        at = (jnp.dot(emt, w[H:], preferred_element_type=jnp.float32)
              + fl_t * gb_ref[l][None, :])
        for q in range(4):
            at = at + jnp.dot(t_refs[l][q] * inv_t, w[q * QQ:(q + 1) * QQ],
                              preferred_element_type=jnp.float32)
        fin = _elu(fin + at)
    fin = _elu(jnp.dot(fin, r0_ref[...],
                       preferred_element_type=jnp.float32) + rb0_ref[...])
    fin = _elu(jnp.dot(fin, r1_ref[...],
                       preferred_element_type=jnp.float32) + rb1_ref[...])
    o_ref[...] = jnp.dot(fin, r2_ref[...],
                         preferred_element_type=jnp.float32) + rb2_ref[...]


def _tc_final(ffp, wf, bf, mt, gw2, gb2, r0, rb0, r1, rb1, r2p, rb2p, ts):
    whole = lambda a: pl.BlockSpec(a.shape, lambda: (0,) * a.ndim)
    args = [ffp, wf, bf, mt, gw2, gb2, r0, rb0, r1, rb1, r2p, rb2p] + list(ts)
    return _pc(
        _tc_final_body,
        in_specs=[whole(a) for a in args],
        out_specs=pl.BlockSpec((FP, D_IN), lambda: (0, 0)),
        out_shape=jax.ShapeDtypeStruct((FP, D_IN), jnp.float32),
    )(*args)


def _sc_mesh():
    return plsc.VectorSubcoreMesh(
        core_axis_name="c", subcore_axis_name="s",
        num_cores=NC, num_subcores=NS)


def _sc_pre_body(y_hbm, yt_hbm, e2_hbm, et2_hbm, z_hbm,
                 es_hbm, st_hbm,
                 acc, acct, rowb, ib, sem):
    c = lax.axis_index("c")
    t = lax.axis_index("s")
    base = t * ROWS_PER_TILE
    pltpu.sync_copy(z_hbm.at[pl.ds(0, ROWS_PER_TILE)],
                    acc.at[pl.ds(base, ROWS_PER_TILE)])

    @pl.when(c == 0)
    def _():
        pltpu.sync_copy(z_hbm.at[pl.ds(0, 64)], acct.at[pl.ds(t * 64, 64)])
    plsc.subcore_barrier()

    rbase = t * (EPR // NS)
    ybase = t * (EPP // NS)
    sel = 1 - c

    def grp(kk, carry):
        pltpu.sync_copy(e2_hbm.at[pl.ds(rbase + kk * GRP, GRP)], ib)
        for j in range(GRP):
            off = ybase + (kk * GRP + j) * CHUNK
            pltpu.sync_copy(y_hbm.at[c].at[pl.ds(off, CHUNK)], rowb)
            pltpu.sync_copy(rowb, acc.at[ib.at[j, sel]], add=True)
        return carry

    lax.fori_loop(0, PREV_GROUPS, grp, 0)
    plsc.subcore_barrier()
    pltpu.sync_copy(acc.at[pl.ds(base, ROWS_PER_TILE)],
                    es_hbm.at[c].at[pl.ds(base, ROWS_PER_TILE)])

    @pl.when(c == 0)
    def _():
        trbase = t * (ETR // NS)
        tybase = t * (ETP // NS)

        def tgrp(kk, carry):
            pltpu.sync_copy(et2_hbm.at[pl.ds(trbase + kk * GRP, GRP)], ib)
            for j in range(GRP):
                off = tybase + (kk * GRP + j) * CHUNK
                pltpu.sync_copy(yt_hbm.at[pl.ds(off, CHUNK)], rowb)
                pltpu.sync_copy(rowb, acct.at[ib.at[j, 1]], add=True)
            return carry

        lax.fori_loop(0, TOF_GROUPS, tgrp, 0)
        plsc.subcore_barrier()
        pltpu.sync_copy(acct.at[pl.ds(t * 64, 64)],
                        st_hbm.at[pl.ds(t * 64, 64)])


def _sc_pre(y, yt, e2, et2, z16):
    return pl.kernel(
        _sc_pre_body,
        compiler_params=pltpu.CompilerParams(use_tc_tiling_on_sc=False),
        out_type=[
            jax.ShapeDtypeStruct((2, NP, DE), jnp.float32),
            jax.ShapeDtypeStruct((FP, DE), jnp.float32),
        ],
        mesh=_sc_mesh(),
        scratch_types=[
            pltpu.VMEM_SHARED((NP, DE), jnp.float32),
            pltpu.VMEM_SHARED((FP, DE), jnp.float32),
            pltpu.VMEM((CHUNK, DE), jnp.float32),
            pltpu.VMEM((GRP, 2, CHUNK), jnp.int32),
            pltpu.SemaphoreType.DMA,
        ],
    )(y, yt, e2, et2, z16)


def _sc_seg_body(x_hbm, e2_hbm, et2_hbm, z_hbm,
                 p_hbm, s_hbm, t_hbm,
                 xs, acc, acct, rows_a, rows_b, ib_a, ib_b, sem_a, sem_b):
    c = lax.axis_index("c")
    t = lax.axis_index("s")
    base = t * ROWS_PER_TILE

    def pipelined_sweep(idx2_hbm, rbase, ngroups, gsel, ssel, accref):
        def issue(ib, rows, sem):
            for j in range(GRP):
                pltpu.async_copy(xs.at[ib.at[j, gsel]], rows.at[j], sem)

        def drain_scatter(ib, rows, sem):
            for j in range(GRP):
                pltpu.make_async_copy(xs.at[ib.at[j, gsel]],
                                      rows.at[j], sem).wait()
                pltpu.sync_copy(rows.at[j], accref.at[ib.at[j, ssel]],
                                add=True)

        pltpu.sync_copy(idx2_hbm.at[pl.ds(rbase, GRP)], ib_a)
        issue(ib_a, rows_a, sem_a)

        def body(kk, carry):
            g1 = rbase + (2 * kk + 1) * GRP
            g2 = rbase + (2 * kk + 2) * GRP
            pltpu.sync_copy(idx2_hbm.at[pl.ds(g1, GRP)], ib_b)
            issue(ib_b, rows_b, sem_b)
            drain_scatter(ib_a, rows_a, sem_a)

            @pl.when(2 * kk + 2 < ngroups)
            def _():
                pltpu.sync_copy(idx2_hbm.at[pl.ds(g2, GRP)], ib_a)
                issue(ib_a, rows_a, sem_a)

            drain_scatter(ib_b, rows_b, sem_b)
            return carry

        lax.fori_loop(0, ngroups // 2, body, 0)

    for r in range(2):
        q = c * 2 + r
        pltpu.sync_copy(x_hbm.at[q].at[pl.ds(base, ROWS_PER_TILE)],
                        xs.at[pl.ds(base, ROWS_PER_TILE)])
        pltpu.sync_copy(z_hbm.at[pl.ds(0, ROWS_PER_TILE)],
                        acc.at[pl.ds(base, ROWS_PER_TILE)])
        pltpu.sync_copy(z_hbm.at[pl.ds(0, 64)], acct.at[pl.ds(t * 64, 64)])
        plsc.subcore_barrier()

        rbase = t * (EPR // NS)
        pipelined_sweep(e2_hbm, rbase, PREV_GROUPS, 0, 1, acc)
        plsc.subcore_barrier()
        pltpu.sync_copy(acc.at[pl.ds(base, ROWS_PER_TILE)],
                        p_hbm.at[q].at[pl.ds(base, ROWS_PER_TILE)])
        pltpu.sync_copy(z_hbm.at[pl.ds(0, ROWS_PER_TILE)],
                        acc.at[pl.ds(base, ROWS_PER_TILE)])
        plsc.subcore_barrier()
        pipelined_sweep(e2_hbm, rbase, PREV_GROUPS, 1, 0, acc)
        plsc.subcore_barrier()
        pltpu.sync_copy(acc.at[pl.ds(base, ROWS_PER_TILE)],
                        s_hbm.at[q].at[pl.ds(base, ROWS_PER_TILE)])
        trbase = t * (ETR // NS)
        pipelined_sweep(et2_hbm, trbase, TOF_GROUPS, 0, 1, acct)
        plsc.subcore_barrier()
        pltpu.sync_copy(acct.at[pl.ds(t * 64, 64)],
                        t_hbm.at[q].at[pl.ds(t * 64, 64)])
        plsc.subcore_barrier()


def _sc_seg(x, e2, et2, z16):
    return pl.kernel(
        _sc_seg_body,
        compiler_params=pltpu.CompilerParams(use_tc_tiling_on_sc=False),
        out_type=[
            jax.ShapeDtypeStruct((4, NP, QQ), jnp.float32),
            jax.ShapeDtypeStruct((4, NP, QQ), jnp.float32),
            jax.ShapeDtypeStruct((4, FP, QQ), jnp.float32),
        ],
        mesh=_sc_mesh(),
        scratch_types=[
            pltpu.VMEM_SHARED((NP, QQ), jnp.float32),
            pltpu.VMEM_SHARED((NP, QQ), jnp.float32),
            pltpu.VMEM_SHARED((FP, QQ), jnp.float32),
            pltpu.VMEM((GRP, CHUNK, QQ), jnp.float32),
            pltpu.VMEM((GRP, CHUNK, QQ), jnp.float32),
            pltpu.VMEM((GRP, 2, CHUNK), jnp.int32),
            pltpu.VMEM((GRP, 2, CHUNK), jnp.int32),
            pltpu.SemaphoreType.DMA,
            pltpu.SemaphoreType.DMA,
        ],
    )(x, e2, et2, z16)


def kernel(instruction_feats, final_feats, instruction_edge_feats,
           to_final_edge_feats, prev_edge_index, to_final_src, to_final_dst,
           W_inst, b_inst, W_final, b_final,
           We_prev, be_prev, We_succ, be_succ, We_tof, be_tof,
           gconv_W, gconv_b,
           rank_W0, rank_b0, rank_W1, rank_b1, rank_W2, rank_b2):
    f32 = jnp.float32
    ifp = jnp.zeros((NP, D_IN), f32).at[:N_INST].set(instruction_feats)
    ffp = jnp.zeros((FP, D_IN), f32).at[:N_FINAL].set(final_feats)
    ep = jnp.zeros((EPP, DE), f32).at[:E_PREV].set(instruction_edge_feats)
    etp = jnp.zeros((ETP, DE), f32).at[:E_TOF].set(to_final_edge_feats)

    pei = prev_edge_index.astype(jnp.int32)
    ps2 = jnp.full((EPP,), N_INST, jnp.int32).at[:E_PREV].set(pei[0])
    pd2 = jnp.full((EPP,), N_INST, jnp.int32).at[:E_PREV].set(pei[1])
    e2 = jnp.stack([ps2.reshape(EPR, CHUNK), pd2.reshape(EPR, CHUNK)], axis=1)
    ts2 = jnp.zeros((ETP,), jnp.int32).at[:E_TOF].set(
        to_final_src.astype(jnp.int32)).reshape(ETR, CHUNK)
    td2 = jnp.full((ETP,), N_FINAL, jnp.int32).at[:E_TOF].set(
        to_final_dst.astype(jnp.int32)).reshape(ETR, CHUNK)
    et2 = jnp.stack([ts2, td2], axis=1)

    z16 = jnp.zeros((ROWS_PER_TILE, DE), f32)

    x = _enc_inst(ifp, W_inst, b_inst.reshape(1, H))
    y = _enc_edge(ep, We_prev, be_prev.reshape(1, EH),
                  We_succ, be_succ.reshape(1, EH))
    yt = _enc_tof(etp, We_tof, be_tof.reshape(1, EH))
    es, st = _sc_pre(y, yt, e2, et2, z16)
    lane = jnp.arange(128)
    node16 = lane // QQ * QQ
    msel = ((lane[:, None] - node16[None, :] == EH)
            & (node16[:, None] == node16[None, :])).astype(f32)
    rel = lane - node16
    memp = ((rel[:, None] < EH) & (lane[None, :] == lane[:, None])
            ).astype(f32)
    mems = ((rel[:, None] < EH)
            & (lane[None, :] == lane[:, None] + EH)).astype(f32)
    meta5 = _meta_inst(es.reshape(2, NPK, 128), msel, memp, mems)
    mt = _meta_tof(st)

    eye8 = jnp.eye(8, dtype=f32)

    def kron8(blk):
        k = jnp.einsum('ab,...ij->...aibj', eye8, blk)
        return k.reshape(blk.shape[:-2] + (128, 128))

    def blocks16(w):
        return w.reshape(NL, 4, QQ, 4, QQ).transpose(0, 1, 3, 2, 4)

    w0big = kron8(blocks16(gconv_W[:, 0, :H]))
    w1big = kron8(blocks16(gconv_W[:, 1, :H]))
    wb16 = jnp.concatenate([gconv_W[:, 0, H:], gconv_W[:, 1, H:]],
                           axis=1)
    wbbig = kron8(wb16.reshape(NL, 16, 4, QQ).transpose(0, 2, 1, 3))
    btile = jnp.tile(gconv_b[:, :2].reshape(NL, 2, 4, QQ), (1, 1, 1, 8))

    t_list = []
    for l in range(NL):
        p, s, tt = _sc_seg(x, e2, et2, z16)
        t_list.append(tt)
        xk = _tc_layer(x.reshape(4, NPK, 128), p.reshape(4, NPK, 128),
                       s.reshape(4, NPK, 128), meta5,
                       w0big[l], w1big[l], wbbig[l], btile[l])
        x = xk.reshape(4, NP, QQ)

    r2p = jnp.pad(rank_W2, ((0, 0), (0, D_IN - 1)))
    rb2p = jnp.broadcast_to(rank_b2, (1, D_IN))
    out = _tc_final(ffp, W_final, b_final.reshape(1, H), mt,
                    gconv_W[:, 2], gconv_b[:, 2],
                    rank_W0, rank_b0.reshape(1, H),
                    rank_W1, rank_b1.reshape(1, H), r2p, rb2p, t_list)
    return out[:N_FINAL, 0]


# speedup vs baseline: 1.0527x; 1.0527x over previous
import functools

import jax
import jax.numpy as jnp
from jax import lax
from jax.experimental import pallas as pl
from jax.experimental.pallas import tpu as pltpu
from jax.experimental.pallas import tpu_sc as plsc

_pc = pl.pallas_call

N_INST = 50000
N_FINAL = 1000
E_PREV = 800000
E_TOF = 50000
D_IN = 128
H = 64
HH = 32
QQ = 16
DE = 16
EH = 8
NL = 11

NP = 50176
FP = 1024
EPP = 819200
EPR = EPP // 128
ETP = 65536
ETR = ETP // 128
NPK = NP // 8

NC, NS = 2, 16
ROWS_PER_TILE = NP // NS
CHUNK = 128
GRP = 4
PREV_GROUPS = EPR // (NS * GRP)
TOF_GROUPS = ETR // (NS * GRP)


def _elu(x):
    return jnp.where(x > 0.0, x, jnp.exp(jnp.minimum(x, 0.0)) - 1.0)


def _enc_inst_body(x_ref, w_ref, b_ref, o_ref):
    y = _elu(jnp.dot(x_ref[...], w_ref[...],
                     preferred_element_type=jnp.float32) + b_ref[...])
    for q in range(4):
        o_ref[q] = y[:, q * QQ:(q + 1) * QQ]


def _enc_inst(xp, w, b):
    return _pc(
        _enc_inst_body,
        grid=(NP // 1024,),
        in_specs=[
            pl.BlockSpec((1024, D_IN), lambda i: (i, 0)),
            pl.BlockSpec((D_IN, H), lambda i: (0, 0)),
            pl.BlockSpec((1, H), lambda i: (0, 0)),
        ],
        out_specs=pl.BlockSpec((4, 1024, QQ), lambda i: (0, i, 0)),
        out_shape=jax.ShapeDtypeStruct((4, NP, QQ), jnp.float32),
    )(xp, w, b)


def _enc_edge_body(e_ref, wp_ref, bp_ref, ws_ref, bs_ref, o_ref):
    e = e_ref[...]
    one = jnp.ones((e.shape[0], 1), jnp.float32)
    pad = jnp.zeros((e.shape[0], 7), jnp.float32)
    yp = _elu(jnp.dot(e, wp_ref[...], preferred_element_type=jnp.float32)
              + bp_ref[...])
    ys = _elu(jnp.dot(e, ws_ref[...], preferred_element_type=jnp.float32)
              + bs_ref[...])
    o_ref[0] = jnp.concatenate([yp, one, pad], axis=1)
    o_ref[1] = jnp.concatenate([ys, one, pad], axis=1)


def _enc_edge(ep, wp, bp, ws, bs):
    return _pc(
        _enc_edge_body,
        grid=(EPP // 2048,),
        in_specs=[
            pl.BlockSpec((2048, DE), lambda i: (i, 0)),
            pl.BlockSpec((DE, EH), lambda i: (0, 0)),
            pl.BlockSpec((1, EH), lambda i: (0, 0)),
            pl.BlockSpec((DE, EH), lambda i: (0, 0)),
            pl.BlockSpec((1, EH), lambda i: (0, 0)),
        ],
        out_specs=pl.BlockSpec((2, 2048, DE), lambda i: (0, i, 0)),
        out_shape=jax.ShapeDtypeStruct((2, EPP, DE), jnp.float32),
    )(ep, wp, bp, ws, bs)


def _enc_tof_body(e_ref, w_ref, b_ref, o_ref):
    e = e_ref[...]
    y = _elu(jnp.dot(e, w_ref[...], preferred_element_type=jnp.float32)
             + b_ref[...])
    one = jnp.ones((e.shape[0], 1), jnp.float32)
    pad = jnp.zeros((e.shape[0], 7), jnp.float32)
    o_ref[...] = jnp.concatenate([y, one, pad], axis=1)


def _enc_tof(ep, w, b):
    return _pc(
        _enc_tof_body,
        grid=(ETP // 2048,),
        in_specs=[
            pl.BlockSpec((2048, DE), lambda i: (i, 0)),
            pl.BlockSpec((DE, EH), lambda i: (0, 0)),
            pl.BlockSpec((1, EH), lambda i: (0, 0)),
        ],
        out_specs=pl.BlockSpec((2048, DE), lambda i: (i, 0)),
        out_shape=jax.ShapeDtypeStruct((ETP, DE), jnp.float32),
    )(ep, w, b)


def _meta_body(es_ref, msel_ref, memp_ref, mems_ref, o_ref):
    dot = lambda a, b: jnp.dot(a, b, preferred_element_type=jnp.float32)
    es0, es1 = es_ref[0], es_ref[1]
    cnt0 = dot(es0, msel_ref[...])
    cnt1 = dot(es1, msel_ref[...])
    inv0 = 1.0 / jnp.maximum(cnt0, 1.0)
    inv1 = 1.0 / jnp.maximum(cnt1, 1.0)
    o_ref[0] = inv0
    o_ref[1] = inv1
    o_ref[2] = (cnt0 > 0.0).astype(jnp.float32)
    o_ref[3] = (cnt1 > 0.0).astype(jnp.float32)
    o_ref[4] = dot(es0 * inv0, memp_ref[...]) + dot(es1 * inv1, mems_ref[...])


def _meta_inst(esk, msel, memp, mems):
    return _pc(
        _meta_body,
        grid=(NPK // 1568,),
        in_specs=[
            pl.BlockSpec((2, 1568, 128), lambda i: (0, i, 0)),
            pl.BlockSpec((128, 128), lambda i: (0, 0)),
            pl.BlockSpec((128, 128), lambda i: (0, 0)),
            pl.BlockSpec((128, 128), lambda i: (0, 0)),
        ],
        out_specs=pl.BlockSpec((5, 1568, 128), lambda i: (0, i, 0)),
        out_shape=jax.ShapeDtypeStruct((5, NPK, 128), jnp.float32),
    )(esk, msel, memp, mems)


def _meta_tof_body(st_ref, o_ref):
    cnt = st_ref[:, EH:EH + 1]
    inv = 1.0 / jnp.maximum(cnt, 1.0)
    fl = (cnt > 0.0).astype(jnp.float32)
    z8 = jnp.zeros((cnt.shape[0], 8), jnp.float32)
    z14 = jnp.zeros((cnt.shape[0], 14), jnp.float32)
    o_ref[...] = jnp.concatenate([st_ref[:, :EH] * inv, z8, inv, fl, z14],
                                 axis=1)


def _meta_tof(st):
    return _pc(
        _meta_tof_body,
        grid=(1,),
        in_specs=[pl.BlockSpec((FP, DE), lambda i: (0, 0))],
        out_specs=pl.BlockSpec((FP, HH), lambda i: (0, 0)),
        out_shape=jax.ShapeDtypeStruct((FP, HH), jnp.float32),
    )(st)


def _tc_layer_body(x_ref, p_ref, s_ref, m_ref, w0_ref, w1_ref, wb_ref,
                   bt_ref, o_ref):
    dot = lambda a, b: jnp.dot(a, b, preferred_element_type=jnp.float32)
    invp, invs = m_ref[0], m_ref[1]
    flp, fls = m_ref[2], m_ref[3]
    em2 = m_ref[4]
    pm = [p_ref[qp] * invp for qp in range(4)]
    sm = [s_ref[qp] * invs for qp in range(4)]
    for q in range(4):
        u = dot(em2, wb_ref[q])
        for qp in range(4):
            u = u + dot(pm[qp], w0_ref[qp, q]) + dot(sm[qp], w1_ref[qp, q])
        u = u + flp * bt_ref[0, q][None, :] + fls * bt_ref[1, q][None, :]
        o_ref[q] = _elu(x_ref[q] + 0.5 * u)


def _tc_layer(xk, pk, sk, meta5, w0big, w1big, wbbig, btile):
    return _pc(
        _tc_layer_body,
        grid=(NPK // 1568,),
        in_specs=[
            pl.BlockSpec((4, 1568, 128), lambda i: (0, i, 0)),
            pl.BlockSpec((4, 1568, 128), lambda i: (0, i, 0)),
            pl.BlockSpec((4, 1568, 128), lambda i: (0, i, 0)),
            pl.BlockSpec((5, 1568, 128), lambda i: (0, i, 0)),
            pl.BlockSpec((4, 4, 128, 128), lambda i: (0, 0, 0, 0)),
            pl.BlockSpec((4, 4, 128, 128), lambda i: (0, 0, 0, 0)),
            pl.BlockSpec((4, 128, 128), lambda i: (0, 0, 0)),
            pl.BlockSpec((2, 4, 128), lambda i: (0, 0, 0)),
        ],
        out_specs=pl.BlockSpec((4, 1568, 128), lambda i: (0, i, 0)),
        out_shape=jax.ShapeDtypeStruct((4, NPK, 128), jnp.float32),
    )(xk, pk, sk, meta5, w0big, w1big, wbbig, btile)


def _tc_final_body(ff_ref, wf_ref, bf_ref, mt_ref, gw_ref, gb_ref,
                   r0_ref, rb0_ref, r1_ref, rb1_ref, r2_ref, rb2_ref,
                   *t_refs):
    t_refs, o_ref = t_refs[:-1], t_refs[-1]
    fin = _elu(jnp.dot(ff_ref[...], wf_ref[...],
                       preferred_element_type=jnp.float32) + bf_ref[...])
    m = mt_ref[...]
    emt = m[:, :EH]
    inv_t, fl_t = m[:, 16:17], m[:, 17:18]
    for l in range(NL):
        w = gw_ref[l]
        at = (jnp.dot(emt, w[H:], preferred_element_type=jnp.float32)
              + fl_t * gb_ref[l][None, :])
        for q in range(4):
            at = at + jnp.dot(t_refs[l][q] * inv_t, w[q * QQ:(q + 1) * QQ],
                              preferred_element_type=jnp.float32)
        fin = _elu(fin + at)
    fin = _elu(jnp.dot(fin, r0_ref[...],
                       preferred_element_type=jnp.float32) + rb0_ref[...])
    fin = _elu(jnp.dot(fin, r1_ref[...],
                       preferred_element_type=jnp.float32) + rb1_ref[...])
    o_ref[...] = jnp.dot(fin, r2_ref[...],
                         preferred_element_type=jnp.float32) + rb2_ref[...]


def _tc_final(ffp, wf, bf, mt, gw2, gb2, r0, rb0, r1, rb1, r2p, rb2p, ts):
    whole = lambda a: pl.BlockSpec(a.shape, lambda: (0,) * a.ndim)
    args = [ffp, wf, bf, mt, gw2, gb2, r0, rb0, r1, rb1, r2p, rb2p] + list(ts)
    return _pc(
        _tc_final_body,
        in_specs=[whole(a) for a in args],
        out_specs=pl.BlockSpec((FP, D_IN), lambda: (0, 0)),
        out_shape=jax.ShapeDtypeStruct((FP, D_IN), jnp.float32),
    )(*args)


def _sc_mesh():
    return plsc.VectorSubcoreMesh(
        core_axis_name="c", subcore_axis_name="s",
        num_cores=NC, num_subcores=NS)


def _sc_pre_body(y_hbm, yt_hbm, e2_hbm, et2_hbm, z_hbm,
                 es_hbm, st_hbm,
                 acc, acct, rowb, ib, sem):
    c = lax.axis_index("c")
    t = lax.axis_index("s")
    base = t * ROWS_PER_TILE
    pltpu.sync_copy(z_hbm.at[pl.ds(0, ROWS_PER_TILE)],
                    acc.at[pl.ds(base, ROWS_PER_TILE)])

    @pl.when(c == 0)
    def _():
        pltpu.sync_copy(z_hbm.at[pl.ds(0, 64)], acct.at[pl.ds(t * 64, 64)])
    plsc.subcore_barrier()

    rbase = t * (EPR // NS)
    ybase = t * (EPP // NS)
    sel = 1 - c

    def grp(kk, carry):
        pltpu.sync_copy(e2_hbm.at[pl.ds(rbase + kk * GRP, GRP)], ib)
        for j in range(GRP):
            off = ybase + (kk * GRP + j) * CHUNK
            pltpu.sync_copy(y_hbm.at[c].at[pl.ds(off, CHUNK)], rowb)
            pltpu.sync_copy(rowb, acc.at[ib.at[j, sel]], add=True)
        return carry

    lax.fori_loop(0, PREV_GROUPS, grp, 0)
    plsc.subcore_barrier()
    pltpu.sync_copy(acc.at[pl.ds(base, ROWS_PER_TILE)],
                    es_hbm.at[c].at[pl.ds(base, ROWS_PER_TILE)])

    @pl.when(c == 0)
    def _():
        trbase = t * (ETR // NS)
        tybase = t * (ETP // NS)

        def tgrp(kk, carry):
            pltpu.sync_copy(et2_hbm.at[pl.ds(trbase + kk * GRP, GRP)], ib)
            for j in range(GRP):
                off = tybase + (kk * GRP + j) * CHUNK
                pltpu.sync_copy(yt_hbm.at[pl.ds(off, CHUNK)], rowb)
                pltpu.sync_copy(rowb, acct.at[ib.at[j, 1]], add=True)
            return carry

        lax.fori_loop(0, TOF_GROUPS, tgrp, 0)
        plsc.subcore_barrier()
        pltpu.sync_copy(acct.at[pl.ds(t * 64, 64)],
                        st_hbm.at[pl.ds(t * 64, 64)])


def _sc_pre(y, yt, e2, et2, z16):
    return pl.kernel(
        _sc_pre_body,
        compiler_params=pltpu.CompilerParams(use_tc_tiling_on_sc=False),
        out_type=[
            jax.ShapeDtypeStruct((2, NP, DE), jnp.float32),
            jax.ShapeDtypeStruct((FP, DE), jnp.float32),
        ],
        mesh=_sc_mesh(),
        scratch_types=[
            pltpu.VMEM_SHARED((NP, DE), jnp.float32),
            pltpu.VMEM_SHARED((FP, DE), jnp.float32),
            pltpu.VMEM((CHUNK, DE), jnp.float32),
            pltpu.VMEM((GRP, 2, CHUNK), jnp.int32),
            pltpu.SemaphoreType.DMA,
        ],
    )(y, yt, e2, et2, z16)


def _sc_seg_body(x_hbm, e2_hbm, et2_hbm, z_hbm,
                 p_hbm, s_hbm, t_hbm,
                 xs, acc, acct, rows_a, rows_b, ib_a, ib_b,
                 sem_a, sem_b, ssem_a, ssem_b):
    c = lax.axis_index("c")
    t = lax.axis_index("s")
    base = t * ROWS_PER_TILE

    def pipelined_sweep(idx2_hbm, rbase, ngroups, gsel, ssel, accref):
        def issue(ib, rows, sem):
            for j in range(GRP):
                pltpu.async_copy(xs.at[ib.at[j, gsel]], rows.at[j], sem)

        def scatter(ib, rows, sem, ssem):
            for j in range(GRP):
                pltpu.make_async_copy(xs.at[ib.at[j, gsel]],
                                      rows.at[j], sem).wait()
                pltpu.async_copy(rows.at[j], accref.at[ib.at[j, ssel]],
                                 ssem, add=True)

        def drain(ib, rows, ssem):
            for j in range(GRP):
                pltpu.make_async_copy(rows.at[j], accref.at[ib.at[j, ssel]],
                                      ssem).wait()

        pltpu.sync_copy(idx2_hbm.at[pl.ds(rbase, GRP)], ib_a)
        issue(ib_a, rows_a, sem_a)

        def body(kk, carry):
            g1 = rbase + (2 * kk + 1) * GRP
            g2 = rbase + (2 * kk + 2) * GRP
            pltpu.sync_copy(idx2_hbm.at[pl.ds(g1, GRP)], ib_b)
            issue(ib_b, rows_b, sem_b)
            scatter(ib_a, rows_a, sem_a, ssem_a)

            @pl.when(2 * kk + 2 < ngroups)
            def _():
                drain(ib_a, rows_a, ssem_a)
                pltpu.sync_copy(idx2_hbm.at[pl.ds(g2, GRP)], ib_a)
                issue(ib_a, rows_a, sem_a)

            scatter(ib_b, rows_b, sem_b, ssem_b)

            @pl.when(2 * kk + 2 < ngroups)
            def _():
                drain(ib_b, rows_b, ssem_b)
            return carry

        lax.fori_loop(0, ngroups // 2, body, 0)
        drain(ib_a, rows_a, ssem_a)
        drain(ib_b, rows_b, ssem_b)

    for r in range(2):
        q = c * 2 + r
        pltpu.sync_copy(x_hbm.at[q].at[pl.ds(base, ROWS_PER_TILE)],
                        xs.at[pl.ds(base, ROWS_PER_TILE)])
        pltpu.sync_copy(z_hbm.at[pl.ds(0, ROWS_PER_TILE)],
                        acc.at[pl.ds(base, ROWS_PER_TILE)])
        pltpu.sync_copy(z_hbm.at[pl.ds(0, 64)], acct.at[pl.ds(t * 64, 64)])
        plsc.subcore_barrier()

        rbase = t * (EPR // NS)
        pipelined_sweep(e2_hbm, rbase, PREV_GROUPS, 0, 1, acc)
        plsc.subcore_barrier()
        pltpu.sync_copy(acc.at[pl.ds(base, ROWS_PER_TILE)],
                        p_hbm.at[q].at[pl.ds(base, ROWS_PER_TILE)])
        pltpu.sync_copy(z_hbm.at[pl.ds(0, ROWS_PER_TILE)],
                        acc.at[pl.ds(base, ROWS_PER_TILE)])
        plsc.subcore_barrier()
        pipelined_sweep(e2_hbm, rbase, PREV_GROUPS, 1, 0, acc)
        plsc.subcore_barrier()
        pltpu.sync_copy(acc.at[pl.ds(base, ROWS_PER_TILE)],
                        s_hbm.at[q].at[pl.ds(base, ROWS_PER_TILE)])
        trbase = t * (ETR // NS)
        pipelined_sweep(et2_hbm, trbase, TOF_GROUPS, 0, 1, acct)
        plsc.subcore_barrier()
        pltpu.sync_copy(acct.at[pl.ds(t * 64, 64)],
                        t_hbm.at[q].at[pl.ds(t * 64, 64)])
        plsc.subcore_barrier()


def _sc_seg(x, e2, et2, z16):
    return pl.kernel(
        _sc_seg_body,
        compiler_params=pltpu.CompilerParams(use_tc_tiling_on_sc=False),
        out_type=[
            jax.ShapeDtypeStruct((4, NP, QQ), jnp.float32),
            jax.ShapeDtypeStruct((4, NP, QQ), jnp.float32),
            jax.ShapeDtypeStruct((4, FP, QQ), jnp.float32),
        ],
        mesh=_sc_mesh(),
        scratch_types=[
            pltpu.VMEM_SHARED((NP, QQ), jnp.float32),
            pltpu.VMEM_SHARED((NP, QQ), jnp.float32),
            pltpu.VMEM_SHARED((FP, QQ), jnp.float32),
            pltpu.VMEM((GRP, CHUNK, QQ), jnp.float32),
            pltpu.VMEM((GRP, CHUNK, QQ), jnp.float32),
            pltpu.VMEM((GRP, 2, CHUNK), jnp.int32),
            pltpu.VMEM((GRP, 2, CHUNK), jnp.int32),
            pltpu.SemaphoreType.DMA,
            pltpu.SemaphoreType.DMA,
            pltpu.SemaphoreType.DMA,
            pltpu.SemaphoreType.DMA,
        ],
    )(x, e2, et2, z16)


def kernel(instruction_feats, final_feats, instruction_edge_feats,
           to_final_edge_feats, prev_edge_index, to_final_src, to_final_dst,
           W_inst, b_inst, W_final, b_final,
           We_prev, be_prev, We_succ, be_succ, We_tof, be_tof,
           gconv_W, gconv_b,
           rank_W0, rank_b0, rank_W1, rank_b1, rank_W2, rank_b2):
    f32 = jnp.float32
    ifp = jnp.zeros((NP, D_IN), f32).at[:N_INST].set(instruction_feats)
    ffp = jnp.zeros((FP, D_IN), f32).at[:N_FINAL].set(final_feats)
    ep = jnp.zeros((EPP, DE), f32).at[:E_PREV].set(instruction_edge_feats)
    etp = jnp.zeros((ETP, DE), f32).at[:E_TOF].set(to_final_edge_feats)

    pei = prev_edge_index.astype(jnp.int32)
    ps2 = jnp.full((EPP,), N_INST, jnp.int32).at[:E_PREV].set(pei[0])
    pd2 = jnp.full((EPP,), N_INST, jnp.int32).at[:E_PREV].set(pei[1])
    e2 = jnp.stack([ps2.reshape(EPR, CHUNK), pd2.reshape(EPR, CHUNK)], axis=1)
    ts2 = jnp.zeros((ETP,), jnp.int32).at[:E_TOF].set(
        to_final_src.astype(jnp.int32)).reshape(ETR, CHUNK)
    td2 = jnp.full((ETP,), N_FINAL, jnp.int32).at[:E_TOF].set(
        to_final_dst.astype(jnp.int32)).reshape(ETR, CHUNK)
    et2 = jnp.stack([ts2, td2], axis=1)

    z16 = jnp.zeros((ROWS_PER_TILE, DE), f32)

    x = _enc_inst(ifp, W_inst, b_inst.reshape(1, H))
    y = _enc_edge(ep, We_prev, be_prev.reshape(1, EH),
                  We_succ, be_succ.reshape(1, EH))
    yt = _enc_tof(etp, We_tof, be_tof.reshape(1, EH))
    es, st = _sc_pre(y, yt, e2, et2, z16)
    lane = jnp.arange(128)
    node16 = lane // QQ * QQ
    msel = ((lane[:, None] - node16[None, :] == EH)
            & (node16[:, None] == node16[None, :])).astype(f32)
    rel = lane - node16
    memp = ((rel[:, None] < EH) & (lane[None, :] == lane[:, None])
            ).astype(f32)
    mems = ((rel[:, None] < EH)
            & (lane[None, :] == lane[:, None] + EH)).astype(f32)
    meta5 = _meta_inst(es.reshape(2, NPK, 128), msel, memp, mems)
    mt = _meta_tof(st)

    eye8 = jnp.eye(8, dtype=f32)

    def kron8(blk):
        k = jnp.einsum('ab,...ij->...aibj', eye8, blk)
        return k.reshape(blk.shape[:-2] + (128, 128))

    def blocks16(w):
        return w.reshape(NL, 4, QQ, 4, QQ).transpose(0, 1, 3, 2, 4)

    w0big = kron8(blocks16(gconv_W[:, 0, :H]))
    w1big = kron8(blocks16(gconv_W[:, 1, :H]))
    wb16 = jnp.concatenate([gconv_W[:, 0, H:], gconv_W[:, 1, H:]],
                           axis=1)
    wbbig = kron8(wb16.reshape(NL, 16, 4, QQ).transpose(0, 2, 1, 3))
    btile = jnp.tile(gconv_b[:, :2].reshape(NL, 2, 4, QQ), (1, 1, 1, 8))

    t_list = []
    for l in range(NL):
        p, s, tt = _sc_seg(x, e2, et2, z16)
        t_list.append(tt)
        xk = _tc_layer(x.reshape(4, NPK, 128), p.reshape(4, NPK, 128),
                       s.reshape(4, NPK, 128), meta5,
                       w0big[l], w1big[l], wbbig[l], btile[l])
        x = xk.reshape(4, NP, QQ)

    r2p = jnp.pad(rank_W2, ((0, 0), (0, D_IN - 1)))
    rb2p = jnp.broadcast_to(rank_b2, (1, D_IN))
    out = _tc_final(ffp, W_final, b_final.reshape(1, H), mt,
                    gconv_W[:, 2], gconv_b[:, 2],
                    rank_W0, rank_b0.reshape(1, H),
                    rank_W1, rank_b1.reshape(1, H), r2p, rb2p, t_list)
    return out[:N_FINAL, 0]
